# baseline (device time: 304936 ns/iter reference)
import jax
import jax.numpy as jnp
from jax import lax
from jax.experimental import pallas as pl
from jax.experimental.pallas import tpu as pltpu

N_DEV = 4
M = 4096
D = 4096
STRIPE = M // N_DEV
HALF = D // 2
SEG = 8
SEGR = STRIPE // SEG
AG_SLOT = (2, 0, 1)


def kernel(partial, resid, gamma):
    x = partial.reshape(M, D)
    gamma2 = gamma.reshape(1, D)

    def body(x_ref, resid_ref, gamma_ref, out_ref,
             sendbuf, comm_cw, comm_ccw, fchunk,
             ld_sems, resid_sems, own_store_sem, st_sems,
             send_cw, recv_cw, send_ccw, recv_ccw,
             ag_send_cw, ag_recv_cw, ag_send_ccw, ag_recv_ccw):
        my = lax.axis_index("i")
        right = lax.rem(my + 1, N_DEV)
        left = lax.rem(my + N_DEV - 1, N_DEV)

        barrier_sem = pltpu.get_barrier_semaphore()
        for nbr in (left, right):
            pl.semaphore_signal(
                barrier_sem, inc=1,
                device_id=(nbr,), device_id_type=pl.DeviceIdType.MESH,
            )
        pl.semaphore_wait(barrier_sem, 2)

        cfg = {
            "cw": (comm_cw, send_cw, recv_cw, ag_send_cw, ag_recv_cw,
                   0, right),
            "ccw": (comm_ccw, send_ccw, recv_ccw, ag_send_ccw, ag_recv_ccw,
                    HALF, left),
        }

        def sb_seg(dname, seg):
            coff = cfg[dname][5]
            return sendbuf.at[pl.ds(seg * SEGR, SEGR), pl.ds(coff, HALF)]

        def rs_remote(dname, slot, seg):
            comm, ssem, rsem = cfg[dname][0], cfg[dname][1], cfg[dname][2]
            dev = cfg[dname][6]
            return pltpu.make_async_remote_copy(
                src_ref=sb_seg(dname, seg),
                dst_ref=comm.at[slot, pl.ds(seg * SEGR, SEGR)],
                send_sem=ssem.at[slot, seg],
                recv_sem=rsem.at[slot, seg],
                device_id=(dev,), device_id_type=pl.DeviceIdType.MESH)

        def ag_remote(src, dname, g, seg):
            comm, ssem, rsem = cfg[dname][0], cfg[dname][3], cfg[dname][4]
            dev = cfg[dname][6]
            return pltpu.make_async_remote_copy(
                src_ref=src,
                dst_ref=comm.at[AG_SLOT[g], pl.ds(seg * SEGR, SEGR)],
                send_sem=ssem.at[g, seg],
                recv_sem=rsem.at[g, seg],
                device_id=(dev,), device_id_type=pl.DeviceIdType.MESH)

        def load_half(rc, dname, sem):
            coff = cfg[dname][5]
            c = pltpu.make_async_copy(
                x_ref.at[pl.ds(rc * STRIPE, STRIPE), pl.ds(coff, HALF)],
                fchunk.at[:, pl.ds(coff, HALF)], sem)
            c.start()
            return c

        sends = {}
        ag = {}
        own_c = lax.rem(my + 1, N_DEV)
        resid_lds = [None] * SEG

        def norm_and_ag0(seg):
            resid_lds[seg].wait()
            r0_, r1_ = seg * SEGR, (seg + 1) * SEGR
            yf = sendbuf[r0_:r1_, :].astype(jnp.float32) + fchunk[r0_:r1_, :]
            rms = jnp.sqrt(jnp.mean(yf * yf, axis=-1, keepdims=True) + 1e-6)
            sendbuf[r0_:r1_, :] = (
                (yf / rms) * gamma_ref[...]).astype(jnp.bfloat16)
            for dname in ("cw", "ccw"):
                rd = ag_remote(sb_seg(dname, seg), dname, 0, seg)
                rd.start()
                ag[(0, dname, seg)] = rd

        l0 = load_half(my, "cw", ld_sems.at[0])
        r0 = load_half(lax.rem(my + 2, N_DEV), "ccw", ld_sems.at[1])
        l0.wait()
        for seg in range(SEG):
            s0, s1 = seg * SEGR, (seg + 1) * SEGR
            sendbuf[s0:s1, 0:HALF] = fchunk[s0:s1, 0:HALF].astype(
                jnp.bfloat16)
            rd = rs_remote("cw", 0, seg)
            rd.start()
            sends[(0, "cw", seg)] = rd
        r0.wait()
        for seg in range(SEG):
            s0, s1 = seg * SEGR, (seg + 1) * SEGR
            sendbuf[s0:s1, HALF:D] = fchunk[s0:s1, HALF:D].astype(
                jnp.bfloat16)
            rd = rs_remote("ccw", 0, seg)
            rd.start()
            sends[(0, "ccw", seg)] = rd

        for h in range(N_DEV - 1):
            slot = h % 2
            rc_cw = lax.rem(my - h - 1 + 2 * N_DEV, N_DEV)
            rc_ccw = lax.rem(my + h + 3, N_DEV)
            lds = {"cw": load_half(rc_cw, "cw", ld_sems.at[0]),
                   "ccw": load_half(rc_ccw, "ccw", ld_sems.at[1])}
            waited_ld = {"cw": False, "ccw": False}
            for seg in range(SEG):
                for dname in ("cw", "ccw"):
                    comm, coff = cfg[dname][0], cfg[dname][5]
                    rd = sends[(h, dname, seg)]
                    rd.wait_recv()
                    if not waited_ld[dname]:
                        lds[dname].wait()
                        waited_ld[dname] = True
                    rd.wait_send()
                    s0, s1 = seg * SEGR, (seg + 1) * SEGR
                    sendbuf[s0:s1, coff:coff + HALF] = (
                        comm[slot, s0:s1, :]
                        + fchunk[s0:s1, coff:coff + HALF].astype(
                            jnp.bfloat16))
                    if h < N_DEV - 2:
                        nrd = rs_remote(dname, (h + 1) % 2, seg)
                        nrd.start()
                        sends[(h + 1, dname, seg)] = nrd
                if h == N_DEV - 2:
                    c = pltpu.make_async_copy(
                        resid_ref.at[
                            pl.ds(own_c * STRIPE + seg * SEGR, SEGR)],
                        fchunk.at[pl.ds(seg * SEGR, SEGR)],
                        resid_sems.at[seg])
                    c.start()
                    resid_lds[seg] = c
                    if seg >= 1:
                        norm_and_ag0(seg - 1)
        norm_and_ag0(SEG - 1)
        own_st = pltpu.make_async_copy(
            sendbuf, out_ref.at[pl.ds(own_c * STRIPE, STRIPE)], own_store_sem)
        own_st.start()

        pend_store = {}
        for g in range(N_DEV - 1):
            slot = AG_SLOT[g]
            oc = {"cw": lax.rem(my - g + 2 * N_DEV, N_DEV),
                  "ccw": lax.rem(my + g + 2, N_DEV)}
            for seg in range(SEG):
                for dname, di in (("cw", 0), ("ccw", 1)):
                    comm, coff = cfg[dname][0], cfg[dname][5]
                    rd = ag[(g, dname, seg)]
                    rd.wait_recv()
                    if g < N_DEV - 2:
                        nrd = ag_remote(
                            comm.at[slot, pl.ds(seg * SEGR, SEGR)],
                            dname, g + 1, seg)
                        nrd.start()
                        ag[(g + 1, dname, seg)] = nrd
                    prev = pend_store.get((dname, seg))
                    if prev is not None:
                        prev.wait()
                    stc = pltpu.make_async_copy(
                        comm.at[slot, pl.ds(seg * SEGR, SEGR)],
                        out_ref.at[
                            pl.ds(oc[dname] * STRIPE + seg * SEGR, SEGR),
                            pl.ds(coff, HALF)],
                        st_sems.at[di, seg])
                    stc.start()
                    pend_store[(dname, seg)] = stc

        for v in pend_store.values():
            v.wait()
        own_st.wait()
        for g in range(N_DEV - 1):
            for seg in range(SEG):
                for dname in ("cw", "ccw"):
                    ag[(g, dname, seg)].wait_send()

    return pl.pallas_call(
        body,
        out_shape=jax.ShapeDtypeStruct((M, D), jnp.bfloat16),
        in_specs=[
            pl.BlockSpec(memory_space=pl.ANY),
            pl.BlockSpec(memory_space=pl.ANY),
            pl.BlockSpec(memory_space=pltpu.VMEM),
        ],
        out_specs=pl.BlockSpec(memory_space=pl.ANY),
        scratch_shapes=[
            pltpu.VMEM((STRIPE, D), jnp.bfloat16),
            pltpu.VMEM((3, STRIPE, HALF), jnp.bfloat16),
            pltpu.VMEM((3, STRIPE, HALF), jnp.bfloat16),
            pltpu.VMEM((STRIPE, D), jnp.float32),
            pltpu.SemaphoreType.DMA((2,)),
            pltpu.SemaphoreType.DMA((SEG,)),
            pltpu.SemaphoreType.DMA,
            pltpu.SemaphoreType.DMA((2, SEG)),
            pltpu.SemaphoreType.DMA((2, SEG)),
            pltpu.SemaphoreType.DMA((2, SEG)),
            pltpu.SemaphoreType.DMA((2, SEG)),
            pltpu.SemaphoreType.DMA((2, SEG)),
            pltpu.SemaphoreType.DMA((3, SEG)),
            pltpu.SemaphoreType.DMA((3, SEG)),
            pltpu.SemaphoreType.DMA((3, SEG)),
            pltpu.SemaphoreType.DMA((3, SEG)),
        ],
        compiler_params=pltpu.CompilerParams(
            collective_id=0, vmem_limit_bytes=63 * 1024 * 1024),
    )(x, resid, gamma2)


# device time: 304259 ns/iter; 1.0022x vs baseline; 1.0022x over previous
import jax
import jax.numpy as jnp
from jax import lax
from jax.experimental import pallas as pl
from jax.experimental.pallas import tpu as pltpu

N_DEV = 4
M = 4096
D = 4096
STRIPE = M // N_DEV
HALF = D // 2
SEG = 4
SEGR = STRIPE // SEG
AG_SLOT = (2, 0, 1)


def kernel(partial, resid, gamma):
    x = partial.reshape(M, D)
    gamma2 = gamma.reshape(1, D)

    def body(x_ref, resid_ref, gamma_ref, out_ref,
             sendbuf, comm_cw, comm_ccw, fchunk,
             ld_sems, resid_sems, own_store_sem, st_sems,
             send_cw, recv_cw, send_ccw, recv_ccw,
             ag_send_cw, ag_recv_cw, ag_send_ccw, ag_recv_ccw):
        my = lax.axis_index("i")
        right = lax.rem(my + 1, N_DEV)
        left = lax.rem(my + N_DEV - 1, N_DEV)

        barrier_sem = pltpu.get_barrier_semaphore()
        for nbr in (left, right):
            pl.semaphore_signal(
                barrier_sem, inc=1,
                device_id=(nbr,), device_id_type=pl.DeviceIdType.MESH,
            )
        pl.semaphore_wait(barrier_sem, 2)

        cfg = {
            "cw": (comm_cw, send_cw, recv_cw, ag_send_cw, ag_recv_cw,
                   0, right),
            "ccw": (comm_ccw, send_ccw, recv_ccw, ag_send_ccw, ag_recv_ccw,
                    HALF, left),
        }

        def sb_seg(dname, seg):
            coff = cfg[dname][5]
            return sendbuf.at[pl.ds(seg * SEGR, SEGR), pl.ds(coff, HALF)]

        def rs_remote(dname, slot, seg):
            comm, ssem, rsem = cfg[dname][0], cfg[dname][1], cfg[dname][2]
            dev = cfg[dname][6]
            return pltpu.make_async_remote_copy(
                src_ref=sb_seg(dname, seg),
                dst_ref=comm.at[slot, pl.ds(seg * SEGR, SEGR)],
                send_sem=ssem.at[slot, seg],
                recv_sem=rsem.at[slot, seg],
                device_id=(dev,), device_id_type=pl.DeviceIdType.MESH)

        def ag_remote(src, dname, g, seg):
            comm, ssem, rsem = cfg[dname][0], cfg[dname][3], cfg[dname][4]
            dev = cfg[dname][6]
            return pltpu.make_async_remote_copy(
                src_ref=src,
                dst_ref=comm.at[AG_SLOT[g], pl.ds(seg * SEGR, SEGR)],
                send_sem=ssem.at[g, seg],
                recv_sem=rsem.at[g, seg],
                device_id=(dev,), device_id_type=pl.DeviceIdType.MESH)

        def load_half(rc, dname, sem):
            coff = cfg[dname][5]
            c = pltpu.make_async_copy(
                x_ref.at[pl.ds(rc * STRIPE, STRIPE), pl.ds(coff, HALF)],
                fchunk.at[:, pl.ds(coff, HALF)], sem)
            c.start()
            return c

        sends = {}
        ag = {}
        own_c = lax.rem(my + 1, N_DEV)
        resid_lds = [None] * SEG

        def norm_and_ag0(seg):
            resid_lds[seg].wait()
            r0_, r1_ = seg * SEGR, (seg + 1) * SEGR
            yf = sendbuf[r0_:r1_, :].astype(jnp.float32) + fchunk[r0_:r1_, :]
            rms = jnp.sqrt(jnp.mean(yf * yf, axis=-1, keepdims=True) + 1e-6)
            sendbuf[r0_:r1_, :] = (
                (yf / rms) * gamma_ref[...]).astype(jnp.bfloat16)
            for dname in ("cw", "ccw"):
                rd = ag_remote(sb_seg(dname, seg), dname, 0, seg)
                rd.start()
                ag[(0, dname, seg)] = rd

        l0 = load_half(my, "cw", ld_sems.at[0])
        r0 = load_half(lax.rem(my + 2, N_DEV), "ccw", ld_sems.at[1])
        l0.wait()
        for seg in range(SEG):
            s0, s1 = seg * SEGR, (seg + 1) * SEGR
            sendbuf[s0:s1, 0:HALF] = fchunk[s0:s1, 0:HALF].astype(
                jnp.bfloat16)
            rd = rs_remote("cw", 0, seg)
            rd.start()
            sends[(0, "cw", seg)] = rd
        r0.wait()
        for seg in range(SEG):
            s0, s1 = seg * SEGR, (seg + 1) * SEGR
            sendbuf[s0:s1, HALF:D] = fchunk[s0:s1, HALF:D].astype(
                jnp.bfloat16)
            rd = rs_remote("ccw", 0, seg)
            rd.start()
            sends[(0, "ccw", seg)] = rd

        for h in range(N_DEV - 1):
            slot = h % 2
            rc_cw = lax.rem(my - h - 1 + 2 * N_DEV, N_DEV)
            rc_ccw = lax.rem(my + h + 3, N_DEV)
            lds = {"cw": load_half(rc_cw, "cw", ld_sems.at[0]),
                   "ccw": load_half(rc_ccw, "ccw", ld_sems.at[1])}
            waited_ld = {"cw": False, "ccw": False}
            for seg in range(SEG):
                for dname in ("cw", "ccw"):
                    comm, coff = cfg[dname][0], cfg[dname][5]
                    rd = sends[(h, dname, seg)]
                    rd.wait_recv()
                    if not waited_ld[dname]:
                        lds[dname].wait()
                        waited_ld[dname] = True
                    rd.wait_send()
                    s0, s1 = seg * SEGR, (seg + 1) * SEGR
                    sendbuf[s0:s1, coff:coff + HALF] = (
                        comm[slot, s0:s1, :]
                        + fchunk[s0:s1, coff:coff + HALF].astype(
                            jnp.bfloat16))
                    if h < N_DEV - 2:
                        nrd = rs_remote(dname, (h + 1) % 2, seg)
                        nrd.start()
                        sends[(h + 1, dname, seg)] = nrd
                if h == N_DEV - 2:
                    c = pltpu.make_async_copy(
                        resid_ref.at[
                            pl.ds(own_c * STRIPE + seg * SEGR, SEGR)],
                        fchunk.at[pl.ds(seg * SEGR, SEGR)],
                        resid_sems.at[seg])
                    c.start()
                    resid_lds[seg] = c
                    if seg >= 1:
                        norm_and_ag0(seg - 1)
        norm_and_ag0(SEG - 1)
        own_st = pltpu.make_async_copy(
            sendbuf, out_ref.at[pl.ds(own_c * STRIPE, STRIPE)], own_store_sem)
        own_st.start()

        pend_store = {}
        for g in range(N_DEV - 1):
            slot = AG_SLOT[g]
            oc = {"cw": lax.rem(my - g + 2 * N_DEV, N_DEV),
                  "ccw": lax.rem(my + g + 2, N_DEV)}
            for seg in range(SEG):
                for dname, di in (("cw", 0), ("ccw", 1)):
                    comm, coff = cfg[dname][0], cfg[dname][5]
                    rd = ag[(g, dname, seg)]
                    rd.wait_recv()
                    if g < N_DEV - 2:
                        nrd = ag_remote(
                            comm.at[slot, pl.ds(seg * SEGR, SEGR)],
                            dname, g + 1, seg)
                        nrd.start()
                        ag[(g + 1, dname, seg)] = nrd
                    prev = pend_store.get((dname, seg))
                    if prev is not None:
                        prev.wait()
                    stc = pltpu.make_async_copy(
                        comm.at[slot, pl.ds(seg * SEGR, SEGR)],
                        out_ref.at[
                            pl.ds(oc[dname] * STRIPE + seg * SEGR, SEGR),
                            pl.ds(coff, HALF)],
                        st_sems.at[di, seg])
                    stc.start()
                    pend_store[(dname, seg)] = stc

        for v in pend_store.values():
            v.wait()
        own_st.wait()
        for g in range(N_DEV - 1):
            for seg in range(SEG):
                for dname in ("cw", "ccw"):
                    ag[(g, dname, seg)].wait_send()

    return pl.pallas_call(
        body,
        out_shape=jax.ShapeDtypeStruct((M, D), jnp.bfloat16),
        in_specs=[
            pl.BlockSpec(memory_space=pl.ANY),
            pl.BlockSpec(memory_space=pl.ANY),
            pl.BlockSpec(memory_space=pltpu.VMEM),
        ],
        out_specs=pl.BlockSpec(memory_space=pl.ANY),
        scratch_shapes=[
            pltpu.VMEM((STRIPE, D), jnp.bfloat16),
            pltpu.VMEM((3, STRIPE, HALF), jnp.bfloat16),
            pltpu.VMEM((3, STRIPE, HALF), jnp.bfloat16),
            pltpu.VMEM((STRIPE, D), jnp.float32),
            pltpu.SemaphoreType.DMA((2,)),
            pltpu.SemaphoreType.DMA((SEG,)),
            pltpu.SemaphoreType.DMA,
            pltpu.SemaphoreType.DMA((2, SEG)),
            pltpu.SemaphoreType.DMA((2, SEG)),
            pltpu.SemaphoreType.DMA((2, SEG)),
            pltpu.SemaphoreType.DMA((2, SEG)),
            pltpu.SemaphoreType.DMA((2, SEG)),
            pltpu.SemaphoreType.DMA((3, SEG)),
            pltpu.SemaphoreType.DMA((3, SEG)),
            pltpu.SemaphoreType.DMA((3, SEG)),
            pltpu.SemaphoreType.DMA((3, SEG)),
        ],
        compiler_params=pltpu.CompilerParams(
            collective_id=0, vmem_limit_bytes=63 * 1024 * 1024),
    )(x, resid, gamma2)
